# baseline (speedup 1.0000x reference)
import jax
import jax.numpy as jnp
from jax import lax
from jax.experimental import pallas as pl
from jax.experimental.pallas import tpu as pltpu
from jax.experimental.pallas import tpu_sc as plsc

N_CLS = 1000
N_CTX = 16
CTX_DIM = 512
SEQ = 77
SUF = SEQ - 1 - N_CTX
B = 1024
TOK_PAD = 128
NC = 2
NS = 16
NW = NC * NS
RPW = B // NW
SLOTS = 6


def _run_jobs(jobs, bufs, gsems, osems):
    n = len(jobs)
    gh = [None] * SLOTS
    outh = [None] * SLOTS

    def issue(j):
        b = j % SLOTS
        if outh[b] is not None:
            outh[b].wait()
            outh[b] = None
        src, idxr, _ = jobs[j]
        gh[b] = pltpu.async_copy(src.at[idxr], bufs.at[b], gsems[b])

    for j in range(min(SLOTS, n)):
        issue(j)
    for j in range(n):
        b = j % SLOTS
        gh[b].wait()
        outh[b] = pltpu.async_copy(bufs.at[b], jobs[j][2], osems[b])
        if j + SLOTS < n:
            issue(j + SLOTS)
    for h in outh:
        if h is not None:
            h.wait()


def _body(pre_n, ctx_n, suf_n, pre_p, ctx_p, suf_p, tok_n, tok_p, cls1,
          out, tok_out,
          idx_all, idx_c, idx_s, tok_buf, bufs,
          gs0, gs1, gs2, gs3, gs4, gs5, os0, os1, os2, os3, os4, os5,
          sem_t0, sem_t1):
    wid = lax.axis_index("s") * NC + lax.axis_index("c")
    base = pl.multiple_of(wid * RPW, RPW)
    pltpu.sync_copy(cls1.at[pl.ds(base, RPW)], idx_all)

    for h in range(RPW // 16):
        c = idx_all[pl.ds(16 * h, 16)]
        for s in range(N_CTX):
            idx_c[s, pl.ds(16 * h, 16)] = c * N_CTX + s
        for s in range(SUF):
            idx_s[s, pl.ds(16 * h, 16)] = c + N_CLS * s

    gsems = (gs0, gs1, gs2, gs3, gs4, gs5)
    osems = (os0, os1, os2, os3, os4, os5)
    tsems = (sem_t0, sem_t1)
    gts = [pltpu.async_copy(tok.at[idx_all], tok_buf.at[p], tsems[p])
           for p, tok in ((0, tok_n), (1, tok_p))]
    jobs = []
    for p, (pre, ctx, suf) in enumerate(
            ((pre_n, ctx_n, suf_n), (pre_p, ctx_p, suf_p))):
        jobs.append((pre, idx_all, out.at[0, p, pl.ds(base, RPW), :]))
        for s in range(N_CTX):
            jobs.append((ctx, idx_c.at[s],
                         out.at[1 + s, p, pl.ds(base, RPW), :]))
        for s in range(SUF):
            jobs.append((suf, idx_s.at[s],
                         out.at[1 + N_CTX + s, p, pl.ds(base, RPW), :]))
    _run_jobs(jobs, bufs, gsems, osems)
    for p in range(2):
        gts[p].wait()
        pltpu.async_copy(tok_buf.at[p], tok_out.at[p, pl.ds(base, RPW), :],
                         tsems[p]).wait()


def kernel(ctx_pos, ctx_neg, token_prefix_pos, token_suffix_pos,
           token_prefix_neg, token_suffix_neg, tokenized_prompts, cls_id):
    pre_n2 = token_prefix_neg.reshape(N_CLS, CTX_DIM)
    pre_p2 = token_prefix_pos.reshape(N_CLS, CTX_DIM)
    ctx_n2 = ctx_neg.reshape(N_CLS * N_CTX, CTX_DIM)
    ctx_p2 = ctx_pos.reshape(N_CLS * N_CTX, CTX_DIM)
    suf_n2 = jnp.transpose(token_suffix_neg, (1, 0, 2)).reshape(
        SUF * N_CLS, CTX_DIM)
    suf_p2 = jnp.transpose(token_suffix_pos, (1, 0, 2)).reshape(
        SUF * N_CLS, CTX_DIM)
    tok_padded = jnp.pad(tokenized_prompts, ((0, 0), (0, TOK_PAD - SEQ)))
    tok_neg = tok_padded[:N_CLS]
    tok_pos = tok_padded[N_CLS:]

    k = pl.kernel(
        _body,
        out_type=(
            jax.ShapeDtypeStruct((SEQ, 2, B, CTX_DIM), jnp.float32),
            jax.ShapeDtypeStruct((2, B, TOK_PAD), jnp.int32),
        ),
        mesh=plsc.VectorSubcoreMesh(core_axis_name="c", subcore_axis_name="s",
                                    num_cores=NC, num_subcores=NS),
        scratch_types=[
            pltpu.VMEM((RPW,), jnp.int32),
            pltpu.VMEM((N_CTX, RPW), jnp.int32),
            pltpu.VMEM((SUF, RPW), jnp.int32),
            pltpu.VMEM((2, RPW, TOK_PAD), jnp.int32),
            pltpu.VMEM((SLOTS, RPW, CTX_DIM), jnp.float32),
            pltpu.SemaphoreType.DMA, pltpu.SemaphoreType.DMA,
            pltpu.SemaphoreType.DMA, pltpu.SemaphoreType.DMA,
            pltpu.SemaphoreType.DMA, pltpu.SemaphoreType.DMA,
            pltpu.SemaphoreType.DMA, pltpu.SemaphoreType.DMA,
            pltpu.SemaphoreType.DMA, pltpu.SemaphoreType.DMA,
            pltpu.SemaphoreType.DMA, pltpu.SemaphoreType.DMA,
            pltpu.SemaphoreType.DMA, pltpu.SemaphoreType.DMA,
        ],
    )
    prompts4, tok3 = k(pre_n2, ctx_n2, suf_n2, pre_p2, ctx_p2, suf_p2,
                       tok_neg, tok_pos, cls_id)
    prompts = jnp.transpose(prompts4, (1, 2, 0, 3)).reshape(
        2 * B, SEQ, CTX_DIM)
    return prompts, tok3.reshape(2 * B, TOK_PAD)[:, :SEQ]


# speedup vs baseline: 1.0142x; 1.0142x over previous
import jax
import jax.numpy as jnp
from jax import lax
from jax.experimental import pallas as pl
from jax.experimental.pallas import tpu as pltpu
from jax.experimental.pallas import tpu_sc as plsc

N_CLS = 1000
N_CTX = 16
CTX_DIM = 512
SEQ = 77
SUF = SEQ - 1 - N_CTX
B = 1024
TOK_PAD = 128
NC = 2
NS = 16
NW = NC * NS
RPW = B // NW
SLOTS = 6


def _run_jobs(jobs, bufs, gsems, osems):
    n = len(jobs)
    gh = [None] * SLOTS
    outh = [None] * SLOTS

    def issue(j):
        b = j % SLOTS
        if outh[b] is not None:
            outh[b].wait()
            outh[b] = None
        src, idxr, _ = jobs[j]
        gh[b] = pltpu.async_copy(src.at[idxr], bufs.at[b], gsems[b])

    for j in range(min(SLOTS, n)):
        issue(j)
    for j in range(n):
        b = j % SLOTS
        gh[b].wait()
        outh[b] = pltpu.async_copy(bufs.at[b], jobs[j][2], osems[b])
        if j + SLOTS < n:
            issue(j + SLOTS)
    for h in outh:
        if h is not None:
            h.wait()


def _body(pre_n, ctx_n, suf_n, pre_p, ctx_p, suf_p, tok_n, tok_p, cls1,
          out, tok_out,
          idx_all, idx_c, idx_s, tok_buf, bufs,
          gs0, gs1, gs2, gs3, gs4, gs5, os0, os1, os2, os3, os4, os5,
          sem_t0, sem_t1):
    wid = lax.axis_index("s") * NC + lax.axis_index("c")
    base = pl.multiple_of(wid * RPW, RPW)
    pltpu.sync_copy(cls1.at[pl.ds(base, RPW)], idx_all)

    for h in range(RPW // 16):
        c = idx_all[pl.ds(16 * h, 16)]
        for s in range(N_CTX):
            idx_c[s, pl.ds(16 * h, 16)] = c * N_CTX + s
        for s in range(SUF):
            idx_s[s, pl.ds(16 * h, 16)] = c + N_CLS * s

    gsems = (gs0, gs1, gs2, gs3, gs4, gs5)
    osems = (os0, os1, os2, os3, os4, os5)
    tsems = (sem_t0, sem_t1)
    gts = [pltpu.async_copy(tok.at[idx_all], tok_buf.at[p], tsems[p])
           for p, tok in ((0, tok_n), (1, tok_p))]
    jobs = []
    for p, (pre, ctx, suf) in enumerate(
            ((pre_n, ctx_n, suf_n), (pre_p, ctx_p, suf_p))):
        pjobs = [(pre, idx_all, out.at[0, p, pl.ds(base, RPW), :])]
        for s in range(N_CTX):
            pjobs.append((ctx, idx_c.at[s],
                          out.at[1 + s, p, pl.ds(base, RPW), :]))
        for s in range(SUF):
            pjobs.append((suf, idx_s.at[s],
                          out.at[1 + N_CTX + s, p, pl.ds(base, RPW), :]))
        jobs.append(pjobs)
    jobs = [j for pair in zip(*jobs) for j in pair]
    _run_jobs(jobs, bufs, gsems, osems)
    for p in range(2):
        gts[p].wait()
        pltpu.async_copy(tok_buf.at[p], tok_out.at[p, pl.ds(base, RPW), :],
                         tsems[p]).wait()


def kernel(ctx_pos, ctx_neg, token_prefix_pos, token_suffix_pos,
           token_prefix_neg, token_suffix_neg, tokenized_prompts, cls_id):
    pre_n2 = token_prefix_neg.reshape(N_CLS, CTX_DIM)
    pre_p2 = token_prefix_pos.reshape(N_CLS, CTX_DIM)
    ctx_n2 = ctx_neg.reshape(N_CLS * N_CTX, CTX_DIM)
    ctx_p2 = ctx_pos.reshape(N_CLS * N_CTX, CTX_DIM)
    suf_n2 = jnp.transpose(token_suffix_neg, (1, 0, 2)).reshape(
        SUF * N_CLS, CTX_DIM)
    suf_p2 = jnp.transpose(token_suffix_pos, (1, 0, 2)).reshape(
        SUF * N_CLS, CTX_DIM)
    tok_padded = jnp.pad(tokenized_prompts, ((0, 0), (0, TOK_PAD - SEQ)))
    tok_neg = tok_padded[:N_CLS]
    tok_pos = tok_padded[N_CLS:]

    k = pl.kernel(
        _body,
        out_type=(
            jax.ShapeDtypeStruct((SEQ, 2, B, CTX_DIM), jnp.float32),
            jax.ShapeDtypeStruct((2, B, TOK_PAD), jnp.int32),
        ),
        mesh=plsc.VectorSubcoreMesh(core_axis_name="c", subcore_axis_name="s",
                                    num_cores=NC, num_subcores=NS),
        scratch_types=[
            pltpu.VMEM((RPW,), jnp.int32),
            pltpu.VMEM((N_CTX, RPW), jnp.int32),
            pltpu.VMEM((SUF, RPW), jnp.int32),
            pltpu.VMEM((2, RPW, TOK_PAD), jnp.int32),
            pltpu.VMEM((SLOTS, RPW, CTX_DIM), jnp.float32),
            pltpu.SemaphoreType.DMA, pltpu.SemaphoreType.DMA,
            pltpu.SemaphoreType.DMA, pltpu.SemaphoreType.DMA,
            pltpu.SemaphoreType.DMA, pltpu.SemaphoreType.DMA,
            pltpu.SemaphoreType.DMA, pltpu.SemaphoreType.DMA,
            pltpu.SemaphoreType.DMA, pltpu.SemaphoreType.DMA,
            pltpu.SemaphoreType.DMA, pltpu.SemaphoreType.DMA,
            pltpu.SemaphoreType.DMA, pltpu.SemaphoreType.DMA,
        ],
    )
    prompts4, tok3 = k(pre_n2, ctx_n2, suf_n2, pre_p2, ctx_p2, suf_p2,
                       tok_neg, tok_pos, cls_id)
    prompts = jnp.transpose(prompts4, (1, 2, 0, 3)).reshape(
        2 * B, SEQ, CTX_DIM)
    return prompts, tok3.reshape(2 * B, TOK_PAD)[:, :SEQ]


# speedup vs baseline: 1.0230x; 1.0086x over previous
import jax
import jax.numpy as jnp
from jax import lax
from jax.experimental import pallas as pl
from jax.experimental.pallas import tpu as pltpu
from jax.experimental.pallas import tpu_sc as plsc

N_CLS = 1000
N_CTX = 16
CTX_DIM = 512
SEQ = 77
SUF = SEQ - 1 - N_CTX
B = 1024
TOK_PAD = 128
NC = 2
NS = 16
NW = NC * NS
RPW = B // NW
SLOTS = 3


def _run_jobs(jobs, bufs, gsems, osems):
    n = len(jobs)
    gh = [None] * SLOTS
    outh = [None] * SLOTS

    def issue(j):
        b = j % SLOTS
        if outh[b] is not None:
            for h in outh[b]:
                h.wait()
            outh[b] = None
        src, idxr, nr, _ = jobs[j]
        gh[b] = pltpu.async_copy(src.at[idxr], bufs.at[b, pl.ds(0, nr)],
                                 gsems[b])

    for j in range(min(SLOTS, n)):
        issue(j)
    for j in range(n):
        b = j % SLOTS
        gh[b].wait()
        outh[b] = [pltpu.async_copy(bufs.at[b, pl.ds(r0, nr)], dst, osems[b])
                   for (r0, nr, dst) in jobs[j][3]]
        if j + SLOTS < n:
            issue(j + SLOTS)
    for hs in outh:
        if hs is not None:
            for h in hs:
                h.wait()


def _body(pre_n, ctx_n, suf_n, pre_p, ctx_p, suf_p, tok_n, tok_p, cls1,
          out, tok_out,
          idx_all, idx_c, idx_s, tok_buf, bufs,
          gs0, gs1, gs2, os0, os1, os2, sem_t0, sem_t1):
    wid = lax.axis_index("s") * NC + lax.axis_index("c")
    base = pl.multiple_of(wid * RPW, RPW)
    pltpu.sync_copy(cls1.at[pl.ds(base, RPW)], idx_all)

    for h in range(RPW // 16):
        c = idx_all[pl.ds(16 * h, 16)]
        for s in range(N_CTX):
            idx_c[s, pl.ds(16 * h, 16)] = c * N_CTX + s
        for s in range(SUF):
            idx_s[s // 2, pl.ds((s % 2) * RPW + 16 * h, 16)] = c + N_CLS * s

    gsems = (gs0, gs1, gs2)
    osems = (os0, os1, os2)
    tsems = (sem_t0, sem_t1)
    gts = [pltpu.async_copy(tok.at[idx_all], tok_buf.at[p], tsems[p])
           for p, tok in ((0, tok_n), (1, tok_p))]
    jobs = []
    for p, (pre, ctx, suf) in enumerate(
            ((pre_n, ctx_n, suf_n), (pre_p, ctx_p, suf_p))):
        pjobs = [(pre, idx_all, RPW,
                  [(0, RPW, out.at[0, p, pl.ds(base, RPW), :])])]
        for s in range(N_CTX):
            pjobs.append((ctx, idx_c.at[s], RPW,
                          [(0, RPW, out.at[1 + s, p, pl.ds(base, RPW), :])]))
        for sp in range(SUF // 2):
            pjobs.append((suf, idx_s.at[sp], 2 * RPW, [
                (0, RPW,
                 out.at[1 + N_CTX + 2 * sp, p, pl.ds(base, RPW), :]),
                (RPW, RPW,
                 out.at[2 + N_CTX + 2 * sp, p, pl.ds(base, RPW), :]),
            ]))
        jobs.append(pjobs)
    jobs = [j for pair in zip(*jobs) for j in pair]
    _run_jobs(jobs, bufs, gsems, osems)
    for p in range(2):
        gts[p].wait()
        pltpu.async_copy(tok_buf.at[p], tok_out.at[p, pl.ds(base, RPW), :],
                         tsems[p]).wait()


def kernel(ctx_pos, ctx_neg, token_prefix_pos, token_suffix_pos,
           token_prefix_neg, token_suffix_neg, tokenized_prompts, cls_id):
    pre_n2 = token_prefix_neg.reshape(N_CLS, CTX_DIM)
    pre_p2 = token_prefix_pos.reshape(N_CLS, CTX_DIM)
    ctx_n2 = ctx_neg.reshape(N_CLS * N_CTX, CTX_DIM)
    ctx_p2 = ctx_pos.reshape(N_CLS * N_CTX, CTX_DIM)
    suf_n2 = jnp.transpose(token_suffix_neg, (1, 0, 2)).reshape(
        SUF * N_CLS, CTX_DIM)
    suf_p2 = jnp.transpose(token_suffix_pos, (1, 0, 2)).reshape(
        SUF * N_CLS, CTX_DIM)
    tok_padded = jnp.pad(tokenized_prompts, ((0, 0), (0, TOK_PAD - SEQ)))
    tok_neg = tok_padded[:N_CLS]
    tok_pos = tok_padded[N_CLS:]

    k = pl.kernel(
        _body,
        out_type=(
            jax.ShapeDtypeStruct((SEQ, 2, B, CTX_DIM), jnp.float32),
            jax.ShapeDtypeStruct((2, B, TOK_PAD), jnp.int32),
        ),
        mesh=plsc.VectorSubcoreMesh(core_axis_name="c", subcore_axis_name="s",
                                    num_cores=NC, num_subcores=NS),
        scratch_types=[
            pltpu.VMEM((RPW,), jnp.int32),
            pltpu.VMEM((N_CTX, RPW), jnp.int32),
            pltpu.VMEM((SUF // 2, 2 * RPW), jnp.int32),
            pltpu.VMEM((2, RPW, TOK_PAD), jnp.int32),
            pltpu.VMEM((SLOTS, 2 * RPW, CTX_DIM), jnp.float32),
            pltpu.SemaphoreType.DMA, pltpu.SemaphoreType.DMA,
            pltpu.SemaphoreType.DMA, pltpu.SemaphoreType.DMA,
            pltpu.SemaphoreType.DMA, pltpu.SemaphoreType.DMA,
            pltpu.SemaphoreType.DMA, pltpu.SemaphoreType.DMA,
        ],
    )
    prompts4, tok3 = k(pre_n2, ctx_n2, suf_n2, pre_p2, ctx_p2, suf_p2,
                       tok_neg, tok_pos, cls_id)
    prompts = jnp.transpose(prompts4, (1, 2, 0, 3)).reshape(
        2 * B, SEQ, CTX_DIM)
    return prompts, tok3.reshape(2 * B, TOK_PAD)[:, :SEQ]


# speedup vs baseline: 1.0251x; 1.0021x over previous
import jax
import jax.numpy as jnp
from jax import lax
from jax.experimental import pallas as pl
from jax.experimental.pallas import tpu as pltpu
from jax.experimental.pallas import tpu_sc as plsc

N_CLS = 1000
N_CTX = 16
CTX_DIM = 512
SEQ = 77
SUF = SEQ - 1 - N_CTX
B = 1024
TOK_PAD = 128
NC = 2
NS = 16
NW = NC * NS
RPW = B // NW
SLOTS = 3


def _run_jobs(jobs, bufs, gsems, osems):
    n = len(jobs)
    gh = [None] * SLOTS
    outh = [None] * SLOTS

    def issue(j):
        b = j % SLOTS
        if outh[b] is not None:
            for h in outh[b]:
                h.wait()
            outh[b] = None
        src, idxr, nr, _ = jobs[j]
        gh[b] = pltpu.async_copy(src.at[idxr], bufs.at[b, pl.ds(0, nr)],
                                 gsems[b])

    for j in range(min(SLOTS, n)):
        issue(j)
    for j in range(n):
        b = j % SLOTS
        gh[b].wait()
        outh[b] = [pltpu.async_copy(bufs.at[b, pl.ds(r0, nr)], dst, osems[b])
                   for (r0, nr, dst) in jobs[j][3]]
        if j + SLOTS < n:
            issue(j + SLOTS)
    for hs in outh:
        if hs is not None:
            for h in hs:
                h.wait()


def _body(pre_n, ctx_n, suf_n, pre_p, ctx_p, suf_p, tok_n, tok_p, cls1,
          out, tok_out,
          idx_all, idx_c, idx_s, tok_buf, bufs,
          gs0, gs1, gs2, os0, os1, os2, sem_t0, sem_t1):
    wid = lax.axis_index("s") * NC + lax.axis_index("c")
    base = pl.multiple_of(wid * RPW, RPW)
    pltpu.sync_copy(cls1.at[pl.ds(base, RPW)], idx_all)

    for h in range(RPW // 16):
        c = idx_all[pl.ds(16 * h, 16)]
        for s in range(N_CTX):
            idx_c[s // 2, pl.ds((s % 2) * RPW + 16 * h, 16)] = c * N_CTX + s
        for s in range(SUF):
            idx_s[s // 2, pl.ds((s % 2) * RPW + 16 * h, 16)] = c + N_CLS * s

    gsems = (gs0, gs1, gs2)
    osems = (os0, os1, os2)
    tsems = (sem_t0, sem_t1)
    gts = [pltpu.async_copy(tok.at[idx_all], tok_buf.at[p], tsems[p])
           for p, tok in ((0, tok_n), (1, tok_p))]
    jobs = []
    for p, (pre, ctx, suf) in enumerate(
            ((pre_n, ctx_n, suf_n), (pre_p, ctx_p, suf_p))):
        pjobs = [(pre, idx_all, RPW,
                  [(0, RPW, out.at[0, p, pl.ds(base, RPW), :])])]
        for sp in range(N_CTX // 2):
            pjobs.append((ctx, idx_c.at[sp], 2 * RPW, [
                (0, RPW, out.at[1 + 2 * sp, p, pl.ds(base, RPW), :]),
                (RPW, RPW, out.at[2 + 2 * sp, p, pl.ds(base, RPW), :]),
            ]))
        for sp in range(SUF // 2):
            pjobs.append((suf, idx_s.at[sp], 2 * RPW, [
                (0, RPW,
                 out.at[1 + N_CTX + 2 * sp, p, pl.ds(base, RPW), :]),
                (RPW, RPW,
                 out.at[2 + N_CTX + 2 * sp, p, pl.ds(base, RPW), :]),
            ]))
        jobs.append(pjobs)
    jobs = [j for pair in zip(*jobs) for j in pair]
    _run_jobs(jobs, bufs, gsems, osems)
    for p in range(2):
        gts[p].wait()
        pltpu.async_copy(tok_buf.at[p], tok_out.at[p, pl.ds(base, RPW), :],
                         tsems[p]).wait()


def kernel(ctx_pos, ctx_neg, token_prefix_pos, token_suffix_pos,
           token_prefix_neg, token_suffix_neg, tokenized_prompts, cls_id):
    pre_n2 = token_prefix_neg.reshape(N_CLS, CTX_DIM)
    pre_p2 = token_prefix_pos.reshape(N_CLS, CTX_DIM)
    ctx_n2 = ctx_neg.reshape(N_CLS * N_CTX, CTX_DIM)
    ctx_p2 = ctx_pos.reshape(N_CLS * N_CTX, CTX_DIM)
    suf_n2 = jnp.transpose(token_suffix_neg, (1, 0, 2)).reshape(
        SUF * N_CLS, CTX_DIM)
    suf_p2 = jnp.transpose(token_suffix_pos, (1, 0, 2)).reshape(
        SUF * N_CLS, CTX_DIM)
    tok_padded = jnp.pad(tokenized_prompts, ((0, 0), (0, TOK_PAD - SEQ)))
    tok_neg = tok_padded[:N_CLS]
    tok_pos = tok_padded[N_CLS:]

    k = pl.kernel(
        _body,
        out_type=(
            jax.ShapeDtypeStruct((SEQ, 2, B, CTX_DIM), jnp.float32),
            jax.ShapeDtypeStruct((2, B, TOK_PAD), jnp.int32),
        ),
        mesh=plsc.VectorSubcoreMesh(core_axis_name="c", subcore_axis_name="s",
                                    num_cores=NC, num_subcores=NS),
        scratch_types=[
            pltpu.VMEM((RPW,), jnp.int32),
            pltpu.VMEM((N_CTX // 2, 2 * RPW), jnp.int32),
            pltpu.VMEM((SUF // 2, 2 * RPW), jnp.int32),
            pltpu.VMEM((2, RPW, TOK_PAD), jnp.int32),
            pltpu.VMEM((SLOTS, 2 * RPW, CTX_DIM), jnp.float32),
            pltpu.SemaphoreType.DMA, pltpu.SemaphoreType.DMA,
            pltpu.SemaphoreType.DMA, pltpu.SemaphoreType.DMA,
            pltpu.SemaphoreType.DMA, pltpu.SemaphoreType.DMA,
            pltpu.SemaphoreType.DMA, pltpu.SemaphoreType.DMA,
        ],
    )
    prompts4, tok3 = k(pre_n2, ctx_n2, suf_n2, pre_p2, ctx_p2, suf_p2,
                       tok_neg, tok_pos, cls_id)
    prompts = jnp.transpose(prompts4, (1, 2, 0, 3)).reshape(
        2 * B, SEQ, CTX_DIM)
    return prompts, tok3.reshape(2 * B, TOK_PAD)[:, :SEQ]
